# baseline (speedup 1.0000x reference)
import jax
import jax.numpy as jnp
from jax.experimental import pallas as pl
from jax.experimental.pallas import tpu as pltpu

N = 8192
B1 = 256
B2 = 16
C1 = 16
C2 = 32
ATT = 128
CLS = 256
F = 800
BN = 256
NB = N // BN


def _stage1_kernel(x_ref, lab_ref, w1b_ref, b1c_ref, w2b_ref, b2c_ref,
                   W1a_ref, b1a_ref, w1o_ref, b1o_ref, se_ref, den_ref):
    bf16 = jnp.bfloat16
    x3 = x_ref[...].astype(bf16)
    x = x3.reshape(BN, 784)
    x1 = jnp.concatenate([x[:, 28 * i:28 * i + 84] for i in range(26)], axis=0)
    hh = jnp.dot(x1, w1b_ref[...],
                 preferred_element_type=jnp.float32).astype(bf16)
    hh = hh.reshape(13, 2, BN, 416)
    hh = jnp.maximum(hh[:, 0], hh[:, 1])
    hh = jnp.maximum(hh[:, :, :208], hh[:, :, 208:])
    hh = jnp.maximum(hh + b1c_ref[...], 0.0)
    x2 = jnp.concatenate([hh[0:11], hh[1:12], hh[2:13]], axis=2)
    gg = jnp.dot(x2.reshape(11 * BN, 624), w2b_ref[...],
                 preferred_element_type=jnp.float32).astype(bf16)
    gg = gg.reshape(11, BN, 352)
    gp = jnp.maximum(gg[0:10].reshape(5, 2, BN, 352)[:, 0],
                     gg[0:10].reshape(5, 2, BN, 352)[:, 1])
    gp = jnp.maximum(gp[:, :, :160], gp[:, :, 160:320])
    gp = jnp.maximum(gp + b2c_ref[...], 0.0)
    g = jnp.concatenate([gp[t] for t in range(5)], axis=1).astype(jnp.float32)
    t1 = jnp.tanh(jnp.dot(g, W1a_ref[...], preferred_element_type=jnp.float32)
                  + b1a_ref[...])
    s = jnp.dot(t1, w1o_ref[...], preferred_element_type=jnp.float32) + b1o_ref[0, 0]
    e = jnp.exp(jax.nn.sigmoid(s))
    wf = (g * e).astype(bf16)
    lab = lab_ref[0, 0, :]
    onehot = (lab[None, :] ==
              jax.lax.broadcasted_iota(jnp.int32, (B1, BN), 0)).astype(bf16)
    part = jnp.dot(onehot, wf, preferred_element_type=jnp.float32)

    @pl.when(pl.program_id(0) == 0)
    def _():
        se_ref[...] = jnp.zeros_like(se_ref)
        den_ref[...] = jnp.zeros_like(den_ref)

    se_ref[...] += part
    den_ref[...] += jnp.sum(e, axis=0, keepdims=True)


def _stage2_kernel(se_ref, den_ref, lab2_ref, W2a_ref, b2a_ref, w2o_ref, b2o_ref,
                   W3a_ref, b3a_ref, w3o_ref, b3o_ref, Wc_ref, bc_ref, wco_ref,
                   bco_ref, out_ref):
    se = se_ref[...] / den_ref[0, 0]
    t2 = jnp.tanh(jnp.dot(se, W2a_ref[...], preferred_element_type=jnp.float32)
                  + b2a_ref[...])
    s2 = jnp.dot(t2, w2o_ref[...], preferred_element_type=jnp.float32) + b2o_ref[0, 0]
    e2 = jnp.exp(jax.nn.sigmoid(s2))
    we = se * e2
    lab2 = lab2_ref[0, 0, :]
    onehot2 = (lab2[None, :] ==
               jax.lax.broadcasted_iota(jnp.int32, (B2, B1), 0)).astype(jnp.float32)
    te = jnp.dot(onehot2, we, preferred_element_type=jnp.float32) / jnp.sum(e2)
    t3 = jnp.tanh(jnp.dot(te, W3a_ref[...], preferred_element_type=jnp.float32)
                  + b3a_ref[...])
    s3 = jnp.dot(t3, w3o_ref[...], preferred_element_type=jnp.float32) + b3o_ref[0, 0]
    e3 = jnp.exp(jax.nn.sigmoid(s3))
    w3 = e3 / jnp.sum(e3)
    outer = jnp.sum(te * w3, axis=0, keepdims=True)
    c = jnp.dot(outer, Wc_ref[...], preferred_element_type=jnp.float32) + bc_ref[...]
    out = jax.nn.sigmoid(jnp.dot(c, wco_ref[...], preferred_element_type=jnp.float32)
                         + bco_ref[0, 0])
    out_ref[...] = out


import numpy as _np

_S1 = _np.zeros((3, 28, 26), _np.float32)
for _d in range(3):
    for _j in range(26):
        _S1[_d, _j + _d, _j] = 1.0
_S2 = _np.zeros((3, 13, 11), _np.float32)
for _d in range(3):
    for _j in range(11):
        _S2[_d, _j + _d, _j] = 1.0


def _band_weights(conv1_k, conv2_k):
    k1 = conv1_k[:, :, 0, :]
    w1b0 = jnp.einsum('dcj,rdk->rcjk', jnp.asarray(_S1), k1)
    w1b = jnp.concatenate([w1b0[:, :, 0::2, :], w1b0[:, :, 1::2, :]],
                          axis=2).reshape(84, 416)
    w2b0 = jnp.einsum('dcj,rdio->rcijo', jnp.asarray(_S2), conv2_k)
    w2b = jnp.concatenate([w2b0[:, :, :, 0:10:2, :], w2b0[:, :, :, 1:10:2, :],
                           w2b0[:, :, :, 10:, :]], axis=3).reshape(624, 352)
    return w1b, w2b


def kernel(x, first_lab, second_lab, conv1_k, conv1_b, conv2_k, conv2_b,
           W1a, b1a, w1o, b1o, W2a, b2a, w2o, b2o, W3a, b3a, w3o, b3o,
           Wc, bc, wco, bco):
    f32 = jnp.float32
    x3 = x.reshape(N, 28, 28)
    labs = first_lab.reshape(NB, 1, BN)
    lab2 = second_lab.reshape(1, 1, B1)
    w1b, w2b = _band_weights(conv1_k, conv2_k)
    w1b = w1b.astype(jnp.bfloat16)
    w2b = w2b.astype(jnp.bfloat16)
    b1c = jnp.tile(conv1_b, 13).reshape(1, 208).astype(jnp.bfloat16)
    b2c = jnp.tile(conv2_b, 5).reshape(1, 160).astype(jnp.bfloat16)

    se, den = pl.pallas_call(
        _stage1_kernel,
        grid=(NB,),
        in_specs=[
            pl.BlockSpec((BN, 28, 28), lambda i: (i, 0, 0)),
            pl.BlockSpec((1, 1, BN), lambda i: (i, 0, 0)),
            pl.BlockSpec((84, 416), lambda i: (0, 0)),
            pl.BlockSpec((1, 208), lambda i: (0, 0)),
            pl.BlockSpec((624, 352), lambda i: (0, 0)),
            pl.BlockSpec((1, 160), lambda i: (0, 0)),
            pl.BlockSpec((F, ATT), lambda i: (0, 0)),
            pl.BlockSpec((1, ATT), lambda i: (0, 0)),
            pl.BlockSpec((ATT, 1), lambda i: (0, 0)),
            pl.BlockSpec((1, 1), lambda i: (0, 0)),
        ],
        out_specs=[
            pl.BlockSpec((B1, F), lambda i: (0, 0)),
            pl.BlockSpec((1, 1), lambda i: (0, 0)),
        ],
        out_shape=[
            jax.ShapeDtypeStruct((B1, F), f32),
            jax.ShapeDtypeStruct((1, 1), f32),
        ],
        compiler_params=pltpu.CompilerParams(dimension_semantics=("arbitrary",)),
    )(x3, labs, w1b, b1c, w2b, b2c, W1a, b1a.reshape(1, ATT), w1o,
      b1o.reshape(1, 1))

    pred = pl.pallas_call(
        _stage2_kernel,
        out_shape=jax.ShapeDtypeStruct((1, 1), f32),
    )(se, den, lab2, W2a, b2a.reshape(1, ATT), w2o, b2o.reshape(1, 1),
      W3a, b3a.reshape(1, ATT), w3o, b3o.reshape(1, 1), Wc,
      bc.reshape(1, CLS), wco, bco.reshape(1, 1))
    return pred


# speedup vs baseline: 1.1117x; 1.1117x over previous
import jax
import jax.numpy as jnp
from jax.experimental import pallas as pl
from jax.experimental.pallas import tpu as pltpu

N = 8192
B1 = 256
B2 = 16
C1 = 16
C2 = 32
ATT = 128
CLS = 256
F = 800
BN = 512
NB = N // BN


def _stage1_kernel(x_ref, lab_ref, w1b_ref, b1c_ref, w2b_ref, b2c_ref,
                   W1a_ref, b1a_ref, w1o_ref, b1o_ref, se_ref, den_ref):
    bf16 = jnp.bfloat16
    x3 = x_ref[...].astype(bf16)
    x = x3.reshape(BN, 784)
    x1 = jnp.concatenate([x[:, 28 * i:28 * i + 84] for i in range(26)], axis=0)
    hh = jnp.dot(x1, w1b_ref[...],
                 preferred_element_type=jnp.float32).astype(bf16)
    hh = hh.reshape(13, 2, BN, 416)
    hh = jnp.maximum(hh[:, 0], hh[:, 1])
    hh = jnp.maximum(hh[:, :, :208], hh[:, :, 208:])
    hh = jnp.maximum(hh + b1c_ref[...], 0.0)
    x2 = jnp.concatenate([hh[0:11], hh[1:12], hh[2:13]], axis=2)
    gg = jnp.dot(x2.reshape(11 * BN, 624), w2b_ref[...],
                 preferred_element_type=jnp.float32).astype(bf16)
    gg = gg.reshape(11, BN, 352)
    gp = jnp.maximum(gg[0:10].reshape(5, 2, BN, 352)[:, 0],
                     gg[0:10].reshape(5, 2, BN, 352)[:, 1])
    gp = jnp.maximum(gp[:, :, :160], gp[:, :, 160:320])
    gp = jnp.maximum(gp + b2c_ref[...], 0.0)
    g = jnp.concatenate([gp[t] for t in range(5)], axis=1).astype(jnp.float32)
    t1 = jnp.tanh(jnp.dot(g, W1a_ref[...], preferred_element_type=jnp.float32)
                  + b1a_ref[...])
    s = jnp.dot(t1, w1o_ref[...], preferred_element_type=jnp.float32) + b1o_ref[0, 0]
    e = jnp.exp(jax.nn.sigmoid(s))
    wf = (g * e).astype(bf16)
    lab = lab_ref[0, 0, :]
    onehot = (lab[None, :] ==
              jax.lax.broadcasted_iota(jnp.int32, (B1, BN), 0)).astype(bf16)
    part = jnp.dot(onehot, wf, preferred_element_type=jnp.float32)

    @pl.when(pl.program_id(0) == 0)
    def _():
        se_ref[...] = jnp.zeros_like(se_ref)
        den_ref[...] = jnp.zeros_like(den_ref)

    se_ref[...] += part
    den_ref[...] += jnp.sum(e, axis=0, keepdims=True)


def _stage2_kernel(se_ref, den_ref, lab2_ref, W2a_ref, b2a_ref, w2o_ref, b2o_ref,
                   W3a_ref, b3a_ref, w3o_ref, b3o_ref, Wc_ref, bc_ref, wco_ref,
                   bco_ref, out_ref):
    se = se_ref[...] / den_ref[0, 0]
    t2 = jnp.tanh(jnp.dot(se, W2a_ref[...], preferred_element_type=jnp.float32)
                  + b2a_ref[...])
    s2 = jnp.dot(t2, w2o_ref[...], preferred_element_type=jnp.float32) + b2o_ref[0, 0]
    e2 = jnp.exp(jax.nn.sigmoid(s2))
    we = se * e2
    lab2 = lab2_ref[0, 0, :]
    onehot2 = (lab2[None, :] ==
               jax.lax.broadcasted_iota(jnp.int32, (B2, B1), 0)).astype(jnp.float32)
    te = jnp.dot(onehot2, we, preferred_element_type=jnp.float32) / jnp.sum(e2)
    t3 = jnp.tanh(jnp.dot(te, W3a_ref[...], preferred_element_type=jnp.float32)
                  + b3a_ref[...])
    s3 = jnp.dot(t3, w3o_ref[...], preferred_element_type=jnp.float32) + b3o_ref[0, 0]
    e3 = jnp.exp(jax.nn.sigmoid(s3))
    w3 = e3 / jnp.sum(e3)
    outer = jnp.sum(te * w3, axis=0, keepdims=True)
    c = jnp.dot(outer, Wc_ref[...], preferred_element_type=jnp.float32) + bc_ref[...]
    out = jax.nn.sigmoid(jnp.dot(c, wco_ref[...], preferred_element_type=jnp.float32)
                         + bco_ref[0, 0])
    out_ref[...] = out


import numpy as _np

_S1 = _np.zeros((3, 28, 26), _np.float32)
for _d in range(3):
    for _j in range(26):
        _S1[_d, _j + _d, _j] = 1.0
_S2 = _np.zeros((3, 13, 11), _np.float32)
for _d in range(3):
    for _j in range(11):
        _S2[_d, _j + _d, _j] = 1.0


def _band_weights(conv1_k, conv2_k):
    k1 = conv1_k[:, :, 0, :]
    w1b0 = jnp.einsum('dcj,rdk->rcjk', jnp.asarray(_S1), k1)
    w1b = jnp.concatenate([w1b0[:, :, 0::2, :], w1b0[:, :, 1::2, :]],
                          axis=2).reshape(84, 416)
    w2b0 = jnp.einsum('dcj,rdio->rcijo', jnp.asarray(_S2), conv2_k)
    w2b = jnp.concatenate([w2b0[:, :, :, 0:10:2, :], w2b0[:, :, :, 1:10:2, :],
                           w2b0[:, :, :, 10:, :]], axis=3).reshape(624, 352)
    return w1b, w2b


def kernel(x, first_lab, second_lab, conv1_k, conv1_b, conv2_k, conv2_b,
           W1a, b1a, w1o, b1o, W2a, b2a, w2o, b2o, W3a, b3a, w3o, b3o,
           Wc, bc, wco, bco):
    f32 = jnp.float32
    x3 = x.reshape(N, 28, 28)
    labs = first_lab.reshape(NB, 1, BN)
    lab2 = second_lab.reshape(1, 1, B1)
    w1b, w2b = _band_weights(conv1_k, conv2_k)
    w1b = w1b.astype(jnp.bfloat16)
    w2b = w2b.astype(jnp.bfloat16)
    b1c = jnp.tile(conv1_b, 13).reshape(1, 208).astype(jnp.bfloat16)
    b2c = jnp.tile(conv2_b, 5).reshape(1, 160).astype(jnp.bfloat16)

    se, den = pl.pallas_call(
        _stage1_kernel,
        grid=(NB,),
        in_specs=[
            pl.BlockSpec((BN, 28, 28), lambda i: (i, 0, 0)),
            pl.BlockSpec((1, 1, BN), lambda i: (i, 0, 0)),
            pl.BlockSpec((84, 416), lambda i: (0, 0)),
            pl.BlockSpec((1, 208), lambda i: (0, 0)),
            pl.BlockSpec((624, 352), lambda i: (0, 0)),
            pl.BlockSpec((1, 160), lambda i: (0, 0)),
            pl.BlockSpec((F, ATT), lambda i: (0, 0)),
            pl.BlockSpec((1, ATT), lambda i: (0, 0)),
            pl.BlockSpec((ATT, 1), lambda i: (0, 0)),
            pl.BlockSpec((1, 1), lambda i: (0, 0)),
        ],
        out_specs=[
            pl.BlockSpec((B1, F), lambda i: (0, 0)),
            pl.BlockSpec((1, 1), lambda i: (0, 0)),
        ],
        out_shape=[
            jax.ShapeDtypeStruct((B1, F), f32),
            jax.ShapeDtypeStruct((1, 1), f32),
        ],
        compiler_params=pltpu.CompilerParams(dimension_semantics=("arbitrary",)),
    )(x3, labs, w1b, b1c, w2b, b2c, W1a, b1a.reshape(1, ATT), w1o,
      b1o.reshape(1, 1))

    pred = pl.pallas_call(
        _stage2_kernel,
        out_shape=jax.ShapeDtypeStruct((1, 1), f32),
    )(se, den, lab2, W2a, b2a.reshape(1, ATT), w2o, b2o.reshape(1, 1),
      W3a, b3a.reshape(1, ATT), w3o, b3o.reshape(1, 1), Wc,
      bc.reshape(1, CLS), wco, bco.reshape(1, 1))
    return pred
